# Initial kernel scaffold; baseline (speedup 1.0000x reference)
import jax
import jax.numpy as jnp
from jax.experimental import pallas as pl


def kernel(x, embedding):
    raise NotImplementedError("write your pallas kernel here")


# speedup vs baseline: 4.7765x; 4.7765x over previous
import jax
import jax.numpy as jnp
from jax.experimental import pallas as pl
from jax.experimental.pallas import tpu as pltpu

_EMBEDDING_DIM = 64
_NUM_CODES = 1024
_BETA = 0.25


def _vq_kernel(x_ref, emb_ref, embT_ref,
               qst_ref, loss_ref, ppl_ref, enc_ref, idx_ref,
               counts_scr, loss_scr):
    i = pl.program_id(0)
    nsteps = pl.num_programs(0)
    b = x_ref.shape[0]
    n_total = b * nsteps

    x = x_ref[...]
    emb = emb_ref[...]

    esq = jnp.sum(emb * emb, axis=0, keepdims=True)
    dist = esq - 2.0 * jnp.dot(x, emb,
                               preferred_element_type=jnp.float32)
    idx = jnp.argmin(dist, axis=1)

    iota = jax.lax.broadcasted_iota(jnp.int32, (b, _NUM_CODES), 1)
    onehot = (iota == idx[:, None]).astype(jnp.float32)
    enc_ref[...] = onehot
    idx_ref[...] = idx.reshape(b, 1)

    quant = jnp.dot(onehot, embT_ref[...],
                    preferred_element_type=jnp.float32)
    qst_ref[...] = x + (quant - x)

    diff = quant - x
    part = jnp.sum(diff * diff).reshape(1, 1)
    prev_loss = jnp.where(i == 0, 0.0, loss_scr[...])
    loss_scr[...] = prev_loss + part

    prev_counts = jnp.where(i == 0, 0.0, counts_scr[...])
    counts_scr[...] = prev_counts + jnp.sum(onehot, axis=0, keepdims=True)

    @pl.when(i == nsteps - 1)
    def _finalize():
        loss_ref[...] = _BETA * loss_scr[...] / (n_total * _EMBEDDING_DIM)
        avg = counts_scr[...] / n_total
        ent = jnp.sum(avg * jnp.log(avg + 1e-10)).reshape(1, 1)
        ppl_ref[...] = jnp.exp(-ent)


def kernel(x, embedding):
    n = x.size // _EMBEDDING_DIM
    flat = x.reshape(n, _EMBEDDING_DIM)
    emb_t = embedding.T

    block = 1024
    grid = (n // block,)

    qst, loss, ppl, enc, idx = pl.pallas_call(
        _vq_kernel,
        grid=grid,
        in_specs=[
            pl.BlockSpec((block, _EMBEDDING_DIM), lambda i: (i, 0)),
            pl.BlockSpec((_EMBEDDING_DIM, _NUM_CODES), lambda i: (0, 0)),
            pl.BlockSpec((_NUM_CODES, _EMBEDDING_DIM), lambda i: (0, 0)),
        ],
        out_specs=[
            pl.BlockSpec((block, _EMBEDDING_DIM), lambda i: (i, 0)),
            pl.BlockSpec((1, 1), lambda i: (0, 0)),
            pl.BlockSpec((1, 1), lambda i: (0, 0)),
            pl.BlockSpec((block, _NUM_CODES), lambda i: (i, 0)),
            pl.BlockSpec((block, 1), lambda i: (i, 0)),
        ],
        out_shape=[
            jax.ShapeDtypeStruct((n, _EMBEDDING_DIM), jnp.float32),
            jax.ShapeDtypeStruct((1, 1), jnp.float32),
            jax.ShapeDtypeStruct((1, 1), jnp.float32),
            jax.ShapeDtypeStruct((n, _NUM_CODES), jnp.float32),
            jax.ShapeDtypeStruct((n, 1), jnp.int32),
        ],
        scratch_shapes=[
            pltpu.VMEM((1, _NUM_CODES), jnp.float32),
            pltpu.VMEM((1, 1), jnp.float32),
        ],
    )(flat, embedding, emb_t)

    return (qst.reshape(x.shape), loss.reshape(()), ppl.reshape(()),
            enc, idx.reshape(n))


# speedup vs baseline: 5.4709x; 1.1454x over previous
import jax
import jax.numpy as jnp
from jax.experimental import pallas as pl
from jax.experimental.pallas import tpu as pltpu

_EMBEDDING_DIM = 64
_NUM_CODES = 1024
_BETA = 0.25


def _vq_kernel(x_ref, emb_ref, embT_ref,
               qst_ref, loss_ref, ppl_ref, enc_ref, idx_ref,
               counts_scr, loss_scr):
    i = pl.program_id(0)
    nsteps = pl.num_programs(0)
    b = x_ref.shape[0]
    n_total = b * nsteps

    x = x_ref[...]
    emb = emb_ref[...]

    esq = jnp.sum(emb * emb, axis=0, keepdims=True)
    dist = esq - 2.0 * jnp.dot(
        x, emb, preferred_element_type=jnp.float32)

    idx = jnp.argmin(dist, axis=1)
    iota = jax.lax.broadcasted_iota(jnp.int32, (b, _NUM_CODES), 1)
    onehot = (iota == idx[:, None]).astype(jnp.float32)
    enc_ref[...] = onehot
    idx_ref[...] = idx.reshape(b, 1)

    quant = jnp.dot(onehot, embT_ref[...],
                    preferred_element_type=jnp.float32)
    qst_ref[...] = x + (quant - x)

    diff = quant - x
    part = jnp.sum(diff * diff).reshape(1, 1)
    prev_loss = jnp.where(i == 0, 0.0, loss_scr[...])
    loss_scr[...] = prev_loss + part

    prev_counts = jnp.where(i == 0, 0.0, counts_scr[...])
    counts_scr[...] = prev_counts + jnp.sum(onehot, axis=0, keepdims=True)

    @pl.when(i == nsteps - 1)
    def _finalize():
        loss_ref[...] = _BETA * loss_scr[...] / (n_total * _EMBEDDING_DIM)
        avg = counts_scr[...] / n_total
        ent = jnp.sum(avg * jnp.log(avg + 1e-10)).reshape(1, 1)
        ppl_ref[...] = jnp.exp(-ent)


def kernel(x, embedding):
    n = x.size // _EMBEDDING_DIM
    flat = x.reshape(n, _EMBEDDING_DIM)
    emb_t = embedding.T

    block = 2048
    grid = (n // block,)

    qst, loss, ppl, enc, idx = pl.pallas_call(
        _vq_kernel,
        grid=grid,
        in_specs=[
            pl.BlockSpec((block, _EMBEDDING_DIM), lambda i: (i, 0)),
            pl.BlockSpec((_EMBEDDING_DIM, _NUM_CODES), lambda i: (0, 0)),
            pl.BlockSpec((_NUM_CODES, _EMBEDDING_DIM), lambda i: (0, 0)),
        ],
        out_specs=[
            pl.BlockSpec((block, _EMBEDDING_DIM), lambda i: (i, 0)),
            pl.BlockSpec((1, 1), lambda i: (0, 0)),
            pl.BlockSpec((1, 1), lambda i: (0, 0)),
            pl.BlockSpec((block, _NUM_CODES), lambda i: (i, 0)),
            pl.BlockSpec((block, 1), lambda i: (i, 0)),
        ],
        out_shape=[
            jax.ShapeDtypeStruct((n, _EMBEDDING_DIM), jnp.float32),
            jax.ShapeDtypeStruct((1, 1), jnp.float32),
            jax.ShapeDtypeStruct((1, 1), jnp.float32),
            jax.ShapeDtypeStruct((n, _NUM_CODES), jnp.float32),
            jax.ShapeDtypeStruct((n, 1), jnp.int32),
        ],
        scratch_shapes=[
            pltpu.VMEM((1, _NUM_CODES), jnp.float32),
            pltpu.VMEM((1, 1), jnp.float32),
        ],
    )(flat, embedding, emb_t)

    return (qst.reshape(x.shape), loss.reshape(()), ppl.reshape(()),
            enc, idx.reshape(n))


# speedup vs baseline: 5.5551x; 1.0154x over previous
import jax
import jax.numpy as jnp
from jax.experimental import pallas as pl
from jax.experimental.pallas import tpu as pltpu

_EMBEDDING_DIM = 64
_NUM_CODES = 1024
_BETA = 0.25


def _vq_kernel(x_ref, emb_ref,
               qst_ref, loss_ref, ppl_ref, enc_ref, idx_ref,
               counts_scr, loss_scr):
    i = pl.program_id(0)
    nsteps = pl.num_programs(0)
    b = x_ref.shape[0]
    n_total = b * nsteps

    x = x_ref[...]
    emb = emb_ref[...]

    esq = jnp.sum(emb * emb, axis=0, keepdims=True)
    xsq = jnp.sum(x * x, axis=1, keepdims=True)
    dist = (xsq + esq) - 2.0 * jnp.dot(
        x, emb, preferred_element_type=jnp.float32)

    idx = jnp.argmin(dist, axis=1)
    iota = jax.lax.broadcasted_iota(jnp.int32, (b, _NUM_CODES), 1)
    onehot = (iota == idx[:, None]).astype(jnp.float32)
    enc_ref[...] = onehot
    idx_ref[...] = idx.reshape(b, 1)

    quant = jax.lax.dot_general(
        onehot, emb, (((1,), (1,)), ((), ())),
        preferred_element_type=jnp.float32)
    qst_ref[...] = x + (quant - x)

    diff = quant - x
    part = jnp.sum(diff * diff).reshape(1, 1)
    prev_loss = jnp.where(i == 0, 0.0, loss_scr[...])
    loss_scr[...] = prev_loss + part

    prev_counts = jnp.where(i == 0, 0.0, counts_scr[...])
    counts_scr[...] = prev_counts + jnp.sum(onehot, axis=0, keepdims=True)

    @pl.when(i == nsteps - 1)
    def _finalize():
        loss_ref[...] = _BETA * loss_scr[...] / (n_total * _EMBEDDING_DIM)
        avg = counts_scr[...] / n_total
        ent = jnp.sum(avg * jnp.log(avg + 1e-10)).reshape(1, 1)
        ppl_ref[...] = jnp.exp(-ent)


def kernel(x, embedding):
    n = x.size // _EMBEDDING_DIM
    flat = x.reshape(n, _EMBEDDING_DIM)

    block = 2048
    grid = (n // block,)

    qst, loss, ppl, enc, idx = pl.pallas_call(
        _vq_kernel,
        grid=grid,
        in_specs=[
            pl.BlockSpec((block, _EMBEDDING_DIM), lambda i: (i, 0)),
            pl.BlockSpec((_EMBEDDING_DIM, _NUM_CODES), lambda i: (0, 0)),
        ],
        out_specs=[
            pl.BlockSpec((block, _EMBEDDING_DIM), lambda i: (i, 0)),
            pl.BlockSpec((1, 1), lambda i: (0, 0)),
            pl.BlockSpec((1, 1), lambda i: (0, 0)),
            pl.BlockSpec((block, _NUM_CODES), lambda i: (i, 0)),
            pl.BlockSpec((block, 1), lambda i: (i, 0)),
        ],
        out_shape=[
            jax.ShapeDtypeStruct((n, _EMBEDDING_DIM), jnp.float32),
            jax.ShapeDtypeStruct((1, 1), jnp.float32),
            jax.ShapeDtypeStruct((1, 1), jnp.float32),
            jax.ShapeDtypeStruct((n, _NUM_CODES), jnp.float32),
            jax.ShapeDtypeStruct((n, 1), jnp.int32),
        ],
        scratch_shapes=[
            pltpu.VMEM((1, _NUM_CODES), jnp.float32),
            pltpu.VMEM((1, 1), jnp.float32),
        ],
    )(flat, embedding)

    return (qst.reshape(x.shape), loss.reshape(()), ppl.reshape(()),
            enc, idx.reshape(n))


# speedup vs baseline: 6.2666x; 1.1281x over previous
import jax
import jax.numpy as jnp
from jax.experimental import pallas as pl
from jax.experimental.pallas import tpu as pltpu

_EMBEDDING_DIM = 64
_NUM_CODES = 1024
_BETA = 0.25


def _vq_kernel(x_ref, emb_ref,
               qst_ref, loss_ref, ppl_ref, enc_ref, idx_ref,
               counts_scr, loss_scr):
    i = pl.program_id(0)
    nsteps = pl.num_programs(0)
    blk_shape = x_ref.shape
    b = blk_shape[0] * blk_shape[1] * blk_shape[2]
    n_total = b * nsteps

    x = x_ref[...].reshape(b, _EMBEDDING_DIM)
    emb = emb_ref[...]

    esq = jnp.sum(emb * emb, axis=0, keepdims=True)
    xsq = jnp.sum(x * x, axis=1, keepdims=True)
    dist = (xsq + esq) - 2.0 * jnp.dot(
        x, emb, preferred_element_type=jnp.float32)

    idx = jnp.argmin(dist, axis=1)
    iota = jax.lax.broadcasted_iota(jnp.int32, (b, _NUM_CODES), 1)
    onehot = (iota == idx[:, None]).astype(jnp.float32)
    enc_ref[...] = onehot
    idx_ref[...] = idx.reshape(idx_ref.shape)

    quant = jax.lax.dot_general(
        onehot, emb, (((1,), (1,)), ((), ())),
        preferred_element_type=jnp.float32)
    qst_ref[...] = (x + (quant - x)).reshape(blk_shape)

    diff = quant - x
    part = jnp.sum(diff * diff).reshape(1, 1)
    prev_loss = jnp.where(i == 0, 0.0, loss_scr[...])
    loss_scr[...] = prev_loss + part

    prev_counts = jnp.where(i == 0, 0.0, counts_scr[...])
    counts_scr[...] = prev_counts + jnp.sum(onehot, axis=0, keepdims=True)

    @pl.when(i == nsteps - 1)
    def _finalize():
        loss_ref[...] = _BETA * loss_scr[...] / (n_total * _EMBEDDING_DIM)
        avg = counts_scr[...] / n_total
        ent = jnp.sum(avg * jnp.log(avg + 1e-10)).reshape(1, 1)
        ppl_ref[...] = jnp.exp(-ent)


def kernel(x, embedding):
    batch, h, w, _ = x.shape
    n = batch * h * w
    imgs_per_blk = 2
    block = imgs_per_blk * h * w
    grid = (batch // imgs_per_blk,)
    idx_rows_per_blk = block // 128

    qst, loss, ppl, enc, idx = pl.pallas_call(
        _vq_kernel,
        grid=grid,
        in_specs=[
            pl.BlockSpec((imgs_per_blk, h, w, _EMBEDDING_DIM),
                         lambda i: (i, 0, 0, 0)),
            pl.BlockSpec((_EMBEDDING_DIM, _NUM_CODES), lambda i: (0, 0)),
        ],
        out_specs=[
            pl.BlockSpec((imgs_per_blk, h, w, _EMBEDDING_DIM),
                         lambda i: (i, 0, 0, 0)),
            pl.BlockSpec((1, 1), lambda i: (0, 0)),
            pl.BlockSpec((1, 1), lambda i: (0, 0)),
            pl.BlockSpec((block, _NUM_CODES), lambda i: (i, 0)),
            pl.BlockSpec((idx_rows_per_blk, 128), lambda i: (i, 0)),
        ],
        out_shape=[
            jax.ShapeDtypeStruct(x.shape, jnp.float32),
            jax.ShapeDtypeStruct((1, 1), jnp.float32),
            jax.ShapeDtypeStruct((1, 1), jnp.float32),
            jax.ShapeDtypeStruct((n, _NUM_CODES), jnp.float32),
            jax.ShapeDtypeStruct((n // 128, 128), jnp.int32),
        ],
        scratch_shapes=[
            pltpu.VMEM((1, _NUM_CODES), jnp.float32),
            pltpu.VMEM((1, 1), jnp.float32),
        ],
    )(x, embedding)

    return (qst, loss.reshape(()), ppl.reshape(()), enc, idx.reshape(n))
